# Initial kernel scaffold; baseline (speedup 1.0000x reference)
import jax
import jax.numpy as jnp
from jax.experimental import pallas as pl


def kernel(input_ids, token_type_ids, word_emb, type_emb, pos_emb, gamma, beta):
    raise NotImplementedError("write your pallas kernel here")


# speedup vs baseline: 5.6705x; 5.6705x over previous
import functools

import jax
import jax.numpy as jnp
from jax import lax
from jax.experimental import pallas as pl
from jax.experimental.pallas import tpu as pltpu
from jax.experimental.pallas import tpu_sc as plsc

NC = 2
NS = 16
NW = NC * NS

EPS = 1e-3
ROWS_PER_STREAM = 128


def _sc_gather(table, idx3d, n_rows):
    H = table.shape[1]
    chunks_per_w = idx3d.shape[1]
    mesh = plsc.VectorSubcoreMesh(core_axis_name="c", subcore_axis_name="s")

    @functools.partial(
        pl.kernel,
        out_type=jax.ShapeDtypeStruct((n_rows, H), jnp.float32),
        mesh=mesh,
        scratch_types=[
            pltpu.VMEM((chunks_per_w, ROWS_PER_STREAM), jnp.int32),
            pltpu.VMEM((ROWS_PER_STREAM, H), jnp.float32),
            pltpu.SemaphoreType.DMA,
        ],
    )
    def k(table_hbm, idx_hbm, out_hbm, idx_v, rows_v, sem):
        wid = lax.axis_index("s") * NC + lax.axis_index("c")
        base = wid * chunks_per_w
        pltpu.sync_copy(idx_hbm.at[wid], idx_v)

        def body(i, carry):
            pltpu.async_copy(table_hbm.at[idx_v.at[i]], rows_v, sem).wait()
            row0 = (base + i) * ROWS_PER_STREAM
            pltpu.sync_copy(rows_v, out_hbm.at[pl.ds(row0, ROWS_PER_STREAM)])
            return carry

        lax.fori_loop(0, chunks_per_w, body, 0)

    return k(table, idx3d)


def _tc_body(g_ref, tt_ref, type_ref, pos_ref, gamma_ref, beta_ref, o_ref):
    x = g_ref[...]
    tt = tt_ref[...].astype(jnp.float32)[..., None]
    t0 = type_ref[0]
    t1 = type_ref[1]
    x = x + t0 + tt * (t1 - t0) + pos_ref[...][None]
    mean = jnp.mean(x, axis=-1, keepdims=True)
    xc = x - mean
    var = jnp.mean(xc * xc, axis=-1, keepdims=True)
    y = xc * lax.rsqrt(var + EPS)
    o_ref[...] = y * gamma_ref[...] + beta_ref[...]


def _tc_add_ln(gathered, token_type_ids, type_emb, pos_slice, gamma, beta):
    B, S = token_type_ids.shape
    H = type_emb.shape[1]
    BB = 8
    grid = (B // BB,)
    return pl.pallas_call(
        _tc_body,
        grid=grid,
        in_specs=[
            pl.BlockSpec((BB, S, H), lambda i: (i, 0, 0)),
            pl.BlockSpec((BB, S), lambda i: (i, 0)),
            pl.BlockSpec((2, H), lambda i: (0, 0)),
            pl.BlockSpec((S, H), lambda i: (0, 0)),
            pl.BlockSpec((1, H), lambda i: (0, 0)),
            pl.BlockSpec((1, H), lambda i: (0, 0)),
        ],
        out_specs=pl.BlockSpec((BB, S, H), lambda i: (i, 0, 0)),
        out_shape=jax.ShapeDtypeStruct((B, S, H), jnp.float32),
        compiler_params=pltpu.CompilerParams(
            dimension_semantics=("arbitrary",)),
    )(gathered, token_type_ids, type_emb, pos_slice, gamma, beta)


def kernel(input_ids, token_type_ids, word_emb, type_emb, pos_emb, gamma, beta):
    B, S = input_ids.shape
    H = word_emb.shape[1]
    n_rows = B * S
    idx3d = input_ids.reshape(NW, n_rows // (NW * ROWS_PER_STREAM),
                              ROWS_PER_STREAM)
    gathered = _sc_gather(word_emb, idx3d, n_rows)
    gathered = gathered.reshape(B, S, H)
    out = _tc_add_ln(gathered, token_type_ids, type_emb, pos_emb[:S],
                     gamma.reshape(1, H), beta.reshape(1, H))
    return out
